# Initial kernel scaffold; baseline (speedup 1.0000x reference)
import jax
import jax.numpy as jnp
from jax.experimental import pallas as pl


def kernel(query_points, support_points, support_features, weights):
    raise NotImplementedError("write your pallas kernel here")


# speedup vs baseline: 2.9356x; 2.9356x over previous
import functools

import jax
import jax.numpy as jnp
import numpy as np
from jax.experimental import pallas as pl
from jax.experimental.pallas import tpu as pltpu

_SIGMA = 0.3
_H = 32
_KP = np.array([
    [0.1, 0.0, 0.0], [-0.1, 0.0, 0.0], [0.0, 0.1, 0.0], [0.0, -0.1, 0.0],
    [0.0, 0.0, 0.1], [0.0, 0.0, -0.1], [0.07, 0.07, 0.0], [0.07, -0.07, 0.0],
    [-0.07, 0.07, 0.0], [-0.07, -0.07, 0.0], [0.07, 0.0, 0.07],
    [-0.07, 0.0, 0.07], [0.0, 0.07, 0.07], [0.0, -0.07, 0.07]],
    dtype=np.float32)
_K = _KP.shape[0]
_KP_BF = np.asarray(_KP.astype(jnp.bfloat16), np.float32)
_C2 = np.sum(_KP * _KP, axis=1)


def _kpconv_block(qpb_ref, q2_ref, s2_ref, spbT_ref, qp_ref, qpcT_ref,
                  f_ref, w_ref, out_ref, *, block_b, c_out):
    mm = jnp.dot(qpb_ref[...], spbT_ref[...],
                 preferred_element_type=jnp.float32)
    knn = (q2_ref[...] - 2.0 * mm) + s2_ref[...]

    col = jax.lax.broadcasted_iota(jnp.int32, knn.shape, 1)
    big = jnp.int32(1 << 30)
    work = knn
    for _ in range(_H):
        m = jnp.min(work, axis=1, keepdims=True)
        idx = jnp.min(jnp.where(work == m, col, big), axis=1, keepdims=True)
        work = jnp.where(col == idx, jnp.inf, work)
    sel = jnp.isinf(work)

    qb = qp_ref[...]
    qpcT = qpcT_ref[...]
    dfx = qpcT[0:1, :] - qb[:, 0:1]
    dfy = qpcT[1:2, :] - qb[:, 1:2]
    dfz = qpcT[2:3, :] - qb[:, 2:3]
    df2 = (dfx * dfx + dfy * dfy) + dfz * dfz
    dfxb = dfx.astype(jnp.bfloat16).astype(jnp.float32)
    dfyb = dfy.astype(jnp.bfloat16).astype(jnp.float32)
    dfzb = dfz.astype(jnp.bfloat16).astype(jnp.float32)

    mind = jnp.full(knn.shape, jnp.inf, jnp.float32)
    am = jnp.zeros(knn.shape, jnp.float32)
    for k in range(_K):
        cx, cy, cz = (float(_KP_BF[k, 0]), float(_KP_BF[k, 1]),
                      float(_KP_BF[k, 2]))
        mk = (dfxb * cx + dfyb * cy) + dfzb * cz
        cur = (df2 - 2.0 * mk) + float(_C2[k])
        upd = cur < mind
        am = jnp.where(upd, jnp.float32(k), am)
        mind = jnp.where(upd, cur, mind)

    dist = jnp.sqrt(jnp.maximum(mind, 1e-12))
    infl = jnp.maximum(1.0 - dist / _SIGMA, 0.0)
    infl = jnp.where(sel, infl, 0.0)

    feats = f_ref[...]
    acc = jnp.zeros((block_b, c_out), jnp.float32)
    for k in range(_K):
        wk = jnp.where(am == jnp.float32(k), infl, 0.0).astype(jnp.bfloat16)
        fk = jnp.dot(wk, feats, preferred_element_type=jnp.float32)
        acc = acc + jnp.dot(fk, w_ref[k], preferred_element_type=jnp.float32)
    out_ref[...] = acc


@jax.jit
def kernel(query_points, support_points, support_features, weights):
    q, _ = query_points.shape
    c_in = support_features.shape[1]
    c_out = weights.shape[2]
    block_b = 128
    n = ((q + block_b - 1) // block_b) * block_b

    pad = n - q
    qp = jnp.pad(query_points, ((0, pad), (0, 0)))
    sp = jnp.pad(support_points, ((0, pad), (0, 0)), constant_values=1e3)
    feats = jnp.pad(support_features, ((0, pad), (0, 0))).astype(jnp.bfloat16)

    qpb = qp.astype(jnp.bfloat16)
    spbT = sp.astype(jnp.bfloat16).T
    q2 = jnp.sum(qp * qp, axis=1, keepdims=True)
    s2 = jnp.sum(sp * sp, axis=1)[None, :]
    qpcT = qp.T

    grid = (n // block_b,)
    out = pl.pallas_call(
        functools.partial(_kpconv_block, block_b=block_b, c_out=c_out),
        grid=grid,
        in_specs=[
            pl.BlockSpec((block_b, 3), lambda i: (i, 0)),
            pl.BlockSpec((block_b, 1), lambda i: (i, 0)),
            pl.BlockSpec((1, n), lambda i: (0, 0)),
            pl.BlockSpec((3, n), lambda i: (0, 0)),
            pl.BlockSpec((block_b, 3), lambda i: (i, 0)),
            pl.BlockSpec((3, n), lambda i: (0, 0)),
            pl.BlockSpec((n, c_in), lambda i: (0, 0)),
            pl.BlockSpec((_K, c_in, c_out), lambda i: (0, 0, 0)),
        ],
        out_specs=pl.BlockSpec((block_b, c_out), lambda i: (i, 0)),
        out_shape=jax.ShapeDtypeStruct((n, c_out), jnp.float32),
        compiler_params=pltpu.CompilerParams(
            dimension_semantics=("arbitrary",)),
    )(qpb, q2, s2, spbT, qp, qpcT, feats, weights)
    return out[:q]


# speedup vs baseline: 3.1818x; 1.0838x over previous
import functools

import jax
import jax.numpy as jnp
import numpy as np
from jax.experimental import pallas as pl
from jax.experimental.pallas import tpu as pltpu

_SIGMA = 0.3
_H = 32
_KP = np.array([
    [0.1, 0.0, 0.0], [-0.1, 0.0, 0.0], [0.0, 0.1, 0.0], [0.0, -0.1, 0.0],
    [0.0, 0.0, 0.1], [0.0, 0.0, -0.1], [0.07, 0.07, 0.0], [0.07, -0.07, 0.0],
    [-0.07, 0.07, 0.0], [-0.07, -0.07, 0.0], [0.07, 0.0, 0.07],
    [-0.07, 0.0, 0.07], [0.0, 0.07, 0.07], [0.0, -0.07, 0.07]],
    dtype=np.float32)
_K = _KP.shape[0]
_KP_BF = np.asarray(_KP.astype(jnp.bfloat16), np.float32)
_C2 = np.sum(_KP * _KP, axis=1)


def _kpconv_block(qpb_ref, q2_ref, s2_ref, spbT_ref, qp_ref, qpcT_ref,
                  f_ref, w_ref, out_ref, *, block_b, c_out):
    mm = jnp.dot(qpb_ref[...], spbT_ref[...],
                 preferred_element_type=jnp.float32)
    knn = (q2_ref[...] - 2.0 * mm) + s2_ref[...]

    col = jax.lax.broadcasted_iota(jnp.int32, knn.shape, 1)
    big = jnp.int32(1 << 30)
    work = knn
    for _ in range(_H):
        m = jnp.min(work, axis=1, keepdims=True)
        idx = jnp.min(jnp.where(work == m, col, big), axis=1, keepdims=True)
        work = jnp.where(col == idx, jnp.inf, work)
    sel = jnp.isinf(work)

    qb = qp_ref[...]
    qpcT = qpcT_ref[...]
    dfx = qpcT[0:1, :] - qb[:, 0:1]
    dfy = qpcT[1:2, :] - qb[:, 1:2]
    dfz = qpcT[2:3, :] - qb[:, 2:3]
    df2 = (dfx * dfx + dfy * dfy) + dfz * dfz
    dfxb = dfx.astype(jnp.bfloat16).astype(jnp.float32)
    dfyb = dfy.astype(jnp.bfloat16).astype(jnp.float32)
    dfzb = dfz.astype(jnp.bfloat16).astype(jnp.float32)

    mind = jnp.full(knn.shape, jnp.inf, jnp.float32)
    am = jnp.zeros(knn.shape, jnp.float32)
    for k in range(_K):
        cx, cy, cz = (float(_KP_BF[k, 0]), float(_KP_BF[k, 1]),
                      float(_KP_BF[k, 2]))
        mk = (dfxb * cx + dfyb * cy) + dfzb * cz
        cur = (df2 - 2.0 * mk) + float(_C2[k])
        upd = cur < mind
        am = jnp.where(upd, jnp.float32(k), am)
        mind = jnp.where(upd, cur, mind)

    dist = jnp.sqrt(jnp.maximum(mind, 1e-12))
    infl = jnp.maximum(1.0 - dist / _SIGMA, 0.0)
    infl = jnp.where(sel, infl, 0.0)

    feats = f_ref[...]
    acc = jnp.zeros((block_b, c_out), jnp.float32)
    for k in range(_K):
        wk = jnp.where(am == jnp.float32(k), infl, 0.0).astype(jnp.bfloat16)
        fk = jnp.dot(wk, feats, preferred_element_type=jnp.float32)
        acc = acc + jnp.dot(fk, w_ref[k], preferred_element_type=jnp.float32)
    out_ref[...] = acc


@jax.jit
def kernel(query_points, support_points, support_features, weights):
    q, _ = query_points.shape
    c_in = support_features.shape[1]
    c_out = weights.shape[2]
    block_b = 256
    n = ((q + block_b - 1) // block_b) * block_b

    pad = n - q
    qp = jnp.pad(query_points, ((0, pad), (0, 0)))
    sp = jnp.pad(support_points, ((0, pad), (0, 0)), constant_values=1e3)
    feats = jnp.pad(support_features, ((0, pad), (0, 0))).astype(jnp.bfloat16)

    qpb = qp.astype(jnp.bfloat16)
    spbT = sp.astype(jnp.bfloat16).T
    q2 = jnp.sum(qp * qp, axis=1, keepdims=True)
    s2 = jnp.sum(sp * sp, axis=1)[None, :]
    qpcT = qp.T

    grid = (n // block_b,)
    out = pl.pallas_call(
        functools.partial(_kpconv_block, block_b=block_b, c_out=c_out),
        grid=grid,
        in_specs=[
            pl.BlockSpec((block_b, 3), lambda i: (i, 0)),
            pl.BlockSpec((block_b, 1), lambda i: (i, 0)),
            pl.BlockSpec((1, n), lambda i: (0, 0)),
            pl.BlockSpec((3, n), lambda i: (0, 0)),
            pl.BlockSpec((block_b, 3), lambda i: (i, 0)),
            pl.BlockSpec((3, n), lambda i: (0, 0)),
            pl.BlockSpec((n, c_in), lambda i: (0, 0)),
            pl.BlockSpec((_K, c_in, c_out), lambda i: (0, 0, 0)),
        ],
        out_specs=pl.BlockSpec((block_b, c_out), lambda i: (i, 0)),
        out_shape=jax.ShapeDtypeStruct((n, c_out), jnp.float32),
        compiler_params=pltpu.CompilerParams(
            dimension_semantics=("parallel",)),
    )(qpb, q2, s2, spbT, qp, qpcT, feats, weights)
    return out[:q]


# speedup vs baseline: 3.4468x; 1.0833x over previous
import functools

import jax
import jax.numpy as jnp
import numpy as np
from jax.experimental import pallas as pl
from jax.experimental.pallas import tpu as pltpu

_SIGMA = 0.3
_H = 32
_KP = np.array([
    [0.1, 0.0, 0.0], [-0.1, 0.0, 0.0], [0.0, 0.1, 0.0], [0.0, -0.1, 0.0],
    [0.0, 0.0, 0.1], [0.0, 0.0, -0.1], [0.07, 0.07, 0.0], [0.07, -0.07, 0.0],
    [-0.07, 0.07, 0.0], [-0.07, -0.07, 0.0], [0.07, 0.0, 0.07],
    [-0.07, 0.0, 0.07], [0.0, 0.07, 0.07], [0.0, -0.07, 0.07]],
    dtype=np.float32)
_K = _KP.shape[0]
_KP_BF = np.asarray(_KP.astype(jnp.bfloat16), np.float32)
_C2 = np.sum(_KP * _KP, axis=1)


def _kpconv_block(qpb_ref, q2_ref, s2_ref, spbT_ref, qp_ref, qpcT_ref,
                  f_ref, w_ref, out_ref, *, block_b, c_out):
    mm = jnp.dot(qpb_ref[...], spbT_ref[...],
                 preferred_element_type=jnp.float32)
    knn = (q2_ref[...] - 2.0 * mm) + s2_ref[...]

    col = jax.lax.broadcasted_iota(jnp.int32, knn.shape, 1)
    big = jnp.int32(1 << 30)
    work = knn
    for _ in range(_H):
        m = jnp.min(work, axis=1, keepdims=True)
        idx = jnp.min(jnp.where(work == m, col, big), axis=1, keepdims=True)
        work = jnp.where(col == idx, jnp.inf, work)
    sel = jnp.isinf(work)

    qb = qp_ref[...]
    qpcT = qpcT_ref[...]
    dfx = qpcT[0:1, :] - qb[:, 0:1]
    dfy = qpcT[1:2, :] - qb[:, 1:2]
    dfz = qpcT[2:3, :] - qb[:, 2:3]
    df2 = (dfx * dfx + dfy * dfy) + dfz * dfz
    dfxb = dfx.astype(jnp.bfloat16).astype(jnp.float32)
    dfyb = dfy.astype(jnp.bfloat16).astype(jnp.float32)
    dfzb = dfz.astype(jnp.bfloat16).astype(jnp.float32)

    a1 = float(2.0 * _KP_BF[0, 0])
    a7 = float(2.0 * _KP_BF[6, 0])
    x2 = dfxb * a1
    y2 = dfyb * a1
    z2 = dfzb * a1
    u2 = dfxb * a7
    v2 = dfyb * a7
    w2 = dfzb * a7
    sks = (x2, -x2, y2, -y2, z2, -z2,
           u2 + v2, u2 - v2, v2 - u2, -(u2 + v2),
           u2 + w2, w2 - u2, v2 + w2, w2 - v2)
    mind = jnp.full(knn.shape, jnp.inf, jnp.float32)
    am = jnp.zeros(knn.shape, jnp.float32)
    for k in range(_K):
        cur = (df2 - sks[k]) + float(_C2[k])
        upd = cur < mind
        am = jnp.where(upd, jnp.float32(k), am)
        mind = jnp.where(upd, cur, mind)

    dist = jnp.sqrt(jnp.maximum(mind, 1e-12))
    infl = jnp.maximum(1.0 - dist / _SIGMA, 0.0)
    inflb = jnp.where(sel, infl, 0.0).astype(jnp.bfloat16)

    feats = f_ref[...]
    acc = jnp.zeros((block_b, c_out), jnp.float32)
    for k in range(_K):
        wk = jnp.where(am == jnp.float32(k), inflb, jnp.bfloat16(0))
        fk = jnp.dot(wk, feats, preferred_element_type=jnp.float32)
        acc = acc + jnp.dot(fk, w_ref[k], preferred_element_type=jnp.float32)
    out_ref[...] = acc


@jax.jit
def kernel(query_points, support_points, support_features, weights):
    q, _ = query_points.shape
    c_in = support_features.shape[1]
    c_out = weights.shape[2]
    block_b = 256
    n = ((q + block_b - 1) // block_b) * block_b

    pad = n - q
    qp = jnp.pad(query_points, ((0, pad), (0, 0)))
    sp = jnp.pad(support_points, ((0, pad), (0, 0)), constant_values=1e3)
    feats = jnp.pad(support_features, ((0, pad), (0, 0))).astype(jnp.bfloat16)

    qpb = qp.astype(jnp.bfloat16)
    spbT = sp.astype(jnp.bfloat16).T
    q2 = jnp.sum(qp * qp, axis=1, keepdims=True)
    s2 = jnp.sum(sp * sp, axis=1)[None, :]
    qpcT = qp.T

    grid = (n // block_b,)
    out = pl.pallas_call(
        functools.partial(_kpconv_block, block_b=block_b, c_out=c_out),
        grid=grid,
        in_specs=[
            pl.BlockSpec((block_b, 3), lambda i: (i, 0)),
            pl.BlockSpec((block_b, 1), lambda i: (i, 0)),
            pl.BlockSpec((1, n), lambda i: (0, 0)),
            pl.BlockSpec((3, n), lambda i: (0, 0)),
            pl.BlockSpec((block_b, 3), lambda i: (i, 0)),
            pl.BlockSpec((3, n), lambda i: (0, 0)),
            pl.BlockSpec((n, c_in), lambda i: (0, 0)),
            pl.BlockSpec((_K, c_in, c_out), lambda i: (0, 0, 0)),
        ],
        out_specs=pl.BlockSpec((block_b, c_out), lambda i: (i, 0)),
        out_shape=jax.ShapeDtypeStruct((n, c_out), jnp.float32),
        compiler_params=pltpu.CompilerParams(
            dimension_semantics=("parallel",)),
    )(qpb, q2, s2, spbT, qp, qpcT, feats, weights)
    return out[:q]
